# baseline (speedup 1.0000x reference)
import jax
import jax.numpy as jnp
from jax import lax
from jax.experimental import pallas as pl
from jax.experimental.pallas import tpu as pltpu
from jax.experimental.pallas import tpu_sc as plsc

N = 10000
NP = 10240
E = 320000
EP = 327680
H = 8
D = 128
NC = 2
NS = 16
HPC = H // NC
EPT = EP // NS
RB = 1024
RPT = NP // NS

_mesh = plsc.VectorSubcoreMesh(core_axis_name="c", subcore_axis_name="s")


def _dense1_body(x_ref, w1_ref, a1_ref, h1_ref, st_ref):
    h1 = jnp.dot(x_ref[...], w1_ref[...], preferred_element_type=jnp.float32)
    h1_ref[...] = h1
    st_ref[...] = jnp.dot(h1, a1_ref[...], preferred_element_type=jnp.float32)


def _dense1(x, W1, A1):
    return pl.pallas_call(
        _dense1_body,
        grid=(NP // RB,),
        in_specs=[pl.BlockSpec((RB, D), lambda i: (i, 0)),
                  pl.BlockSpec((D, D), lambda i: (0, 0)),
                  pl.BlockSpec((D, 2 * H), lambda i: (0, 0))],
        out_specs=[pl.BlockSpec((RB, D), lambda i: (i, 0)),
                   pl.BlockSpec((RB, 2 * H), lambda i: (i, 0))],
        out_shape=[jax.ShapeDtypeStruct((NP, D), jnp.float32),
                   jax.ShapeDtypeStruct((NP, 2 * H), jnp.float32)],
    )(x, W1, A1)


def _dense2_body(agg_ref, b1_ref, w2_ref, a2_ref, hin_ref, st_ref):
    o = agg_ref[0] + agg_ref[1] + b1_ref[...]
    hin = jnp.where(o > 0, o, jnp.exp(o) - 1.0)
    hin_ref[...] = hin
    u2 = jnp.dot(w2_ref[...], a2_ref[...], preferred_element_type=jnp.float32)
    st_ref[...] = jnp.dot(hin, u2, preferred_element_type=jnp.float32)


def _dense2(aggs1, b1, W2, A2):
    return pl.pallas_call(
        _dense2_body,
        grid=(NP // RB,),
        in_specs=[pl.BlockSpec((2, RB, D), lambda i: (0, i, 0)),
                  pl.BlockSpec((1, D), lambda i: (0, 0)),
                  pl.BlockSpec((D, H * D), lambda i: (0, 0)),
                  pl.BlockSpec((H * D, 2 * H), lambda i: (0, 0))],
        out_specs=[pl.BlockSpec((RB, D), lambda i: (i, 0)),
                   pl.BlockSpec((RB, 2 * H), lambda i: (i, 0))],
        out_shape=[jax.ShapeDtypeStruct((NP, D), jnp.float32),
                   jax.ShapeDtypeStruct((NP, 2 * H), jnp.float32)],
    )(aggs1, b1.reshape(1, D), W2, A2)


def _dense3_body(agg_ref, w2s_ref, b2_ref, out_ref):
    acc = jnp.zeros((RB, D), jnp.float32)
    for c in range(NC):
        for p in range(HPC):
            acc = acc + jnp.dot(agg_ref[c, p], w2s_ref[c * HPC + p],
                                preferred_element_type=jnp.float32)
    out_ref[...] = acc * (1.0 / H) + b2_ref[...]


def _dense3(agg2, W2stack, b2):
    return pl.pallas_call(
        _dense3_body,
        grid=(NP // RB,),
        in_specs=[pl.BlockSpec((NC, HPC, RB, D), lambda i: (0, 0, i, 0)),
                  pl.BlockSpec((H, D, D), lambda i: (0, 0, 0)),
                  pl.BlockSpec((1, D), lambda i: (0, 0))],
        out_specs=pl.BlockSpec((RB, D), lambda i: (i, 0)),
        out_shape=jax.ShapeDtypeStruct((NP, D), jnp.float32),
    )(agg2, W2stack, b2.reshape(1, D))


_CH = 2048


def _softmax_body(stt, srcdst, alpha, s_vm, t_vm, d_vm, src_c, dst_c, ex_c,
                  den_sh):
    c = lax.axis_index("c")
    s = lax.axis_index("s")
    head = c * HPC + s // 4
    q = s % 4
    EQ = EP // 4
    base_q = q * EQ

    pltpu.sync_copy(stt.at[head], s_vm)
    pltpu.sync_copy(stt.at[H + head], t_vm)

    zero16 = jnp.zeros((16,), jnp.float32)

    @pl.loop(0, NP // 16)
    def _zero(i):
        d_vm[pl.ds(i * 16, 16)] = zero16

    @pl.loop(0, EQ // _CH)
    def _pass1(ci):
        base = base_q + ci * _CH
        pltpu.sync_copy(srcdst.at[0].at[pl.ds(base, _CH)], src_c)
        pltpu.sync_copy(srcdst.at[1].at[pl.ds(base, _CH)], dst_c)

        @pl.loop(0, _CH // 16)
        def _inner(j):
            sl = pl.ds(j * 16, 16)
            isrc = src_c[sl]
            idst = dst_c[sl]
            e = plsc.load_gather(s_vm, [isrc]) + plsc.load_gather(t_vm, [idst])
            e = jnp.where(e >= 0.0, e, 0.2 * e)
            ex = jnp.exp(e)
            ex_c[sl] = ex
            plsc.addupdate_scatter(d_vm, [idst], ex)

        pltpu.sync_copy(ex_c, alpha.at[head].at[pl.ds(base, _CH)])

    pltpu.sync_copy(d_vm, den_sh.at[s])
    plsc.subcore_barrier()
    g4 = (s // 4) * 4

    pltpu.sync_copy(den_sh.at[g4], d_vm)
    for k in range(1, 4):
        pltpu.sync_copy(den_sh.at[g4 + k], t_vm)

        @pl.loop(0, NP // 16)
        def _acc(i):
            sl = pl.ds(i * 16, 16)
            d_vm[sl] = d_vm[sl] + t_vm[sl]

    @pl.loop(0, NP // 16)
    def _recip(i):
        sl = pl.ds(i * 16, 16)
        d_vm[sl] = 1.0 / (d_vm[sl] + 1e-16)

    @pl.loop(0, EQ // _CH)
    def _pass2(ci):
        base = base_q + ci * _CH
        pltpu.sync_copy(srcdst.at[1].at[pl.ds(base, _CH)], dst_c)
        pltpu.sync_copy(alpha.at[head].at[pl.ds(base, _CH)], ex_c)

        @pl.loop(0, _CH // 16)
        def _inner(j):
            sl = pl.ds(j * 16, 16)
            rd = plsc.load_gather(d_vm, [dst_c[sl]])
            ex_c[sl] = ex_c[sl] * rd

        pltpu.sync_copy(ex_c, alpha.at[head].at[pl.ds(base, _CH)])


def _softmax_sc(stt, srcdst):
    return pl.kernel(
        _softmax_body,
        out_type=jax.ShapeDtypeStruct((H, EP), jnp.float32),
        mesh=_mesh,
        compiler_params=pltpu.CompilerParams(needs_layout_passes=False),
        scratch_types=[
            pltpu.VMEM((NP,), jnp.float32),
            pltpu.VMEM((NP,), jnp.float32),
            pltpu.VMEM((NP,), jnp.float32),
            pltpu.VMEM((_CH,), jnp.int32),
            pltpu.VMEM((_CH,), jnp.int32),
            pltpu.VMEM((_CH,), jnp.float32),
            pltpu.VMEM_SHARED((NS, NP), jnp.float32),
        ],
    )(stt, srcdst)


_CB = 128


def _agg_pipeline(srcdst, alpha_slice, feat, acc_sh, SD, AB, MSG, isem, gsem,
                  ssem, base_t, nch, scale_fn):
    def issue_idx(ci, k):
        base = base_t + ci * _CB
        pltpu.async_copy(srcdst.at[:, pl.ds(base, _CB)], SD[k], isem)
        pltpu.async_copy(alpha_slice(pl.ds(base, _CB)), AB[k], isem)

    def wait_idx(k):
        pltpu.make_async_copy(srcdst.at[:, pl.ds(0, _CB)], SD[k], isem).wait()
        pltpu.make_async_copy(alpha_slice(pl.ds(0, _CB)), AB[k], isem).wait()

    def fire_gather(k2, k4):
        for q in range(4):
            sl = pl.ds(q * 32, 32)
            pltpu.async_copy(feat.at[SD[k4].at[0].at[sl]], MSG[k2].at[sl],
                             gsem)

    def wait_gather(k2, k4):
        for q in range(4):
            sl = pl.ds(q * 32, 32)
            pltpu.make_async_copy(feat.at[SD[k4].at[0].at[sl]],
                                  MSG[k2].at[sl], gsem).wait()

    def fire_scatter(k2, k4):
        pltpu.async_copy(MSG[k2], acc_sh.at[SD[k4].at[1]], ssem, add=True)

    def wait_scatter(k2, k4):
        pltpu.make_async_copy(MSG[k2], acc_sh.at[SD[k4].at[1]], ssem).wait()

    issue_idx(0, 0)
    issue_idx(1, 1)
    wait_idx(0)
    fire_gather(0, 0)

    @pl.loop(0, nch // 4)
    def _grp(g):
        for k in range(4):
            ci = g * 4 + k
            k2 = k % 2

            @pl.when(ci >= 1)
            def _():
                wait_scatter((k - 1) % 2, (k - 1) % 4)

            @pl.when(ci + 2 < nch)
            def _():
                issue_idx(ci + 2, (k + 2) % 4)

            @pl.when(ci + 1 < nch)
            def _():
                wait_idx((k + 1) % 4)
                fire_gather((k + 1) % 2, (k + 1) % 4)

            wait_gather(k2, k)
            scale_fn(MSG[k2], AB[k])
            fire_scatter(k2, k)

    wait_scatter(1, 3)


def _zero_acc_rows(msg, acc_sh, r0):
    zero16 = jnp.zeros((16,), jnp.float32)

    @pl.loop(0, _CB)
    def _z(j):
        for hh in range(8):
            msg[j, pl.ds(hh * 16, 16)] = zero16

    for k in range(RPT // _CB):
        pltpu.sync_copy(msg.at[pl.ds(0, _CB)],
                        acc_sh.at[pl.ds(r0 + k * _CB, _CB)])


_EPW = EP // (NC * NS)


def _agg1_body(srcdst, alpha, h1, aggs, sd0, sd1, sd2, sd3, ab0, ab1, ab2,
               ab3, msg0, msg1, isem, gsem, ssem, acc_sh):
    c = lax.axis_index("c")
    s = lax.axis_index("s")
    r0 = s * RPT
    _zero_acc_rows(msg0, acc_sh, r0)
    plsc.subcore_barrier()

    def scale(msg, ab):
        @pl.loop(0, _CB, unroll=2)
        def _edge(j):
            jj = jnp.full((16,), j, jnp.int32)
            for hh in range(H):
                av = plsc.load_gather(ab, [jnp.full((16,), hh, jnp.int32), jj])
                sl = pl.ds(hh * 16, 16)
                msg[j, sl] = msg[j, sl] * av

    _agg_pipeline(srcdst, lambda ds: alpha.at[:, ds], h1, acc_sh,
                  [sd0, sd1, sd2, sd3], [ab0, ab1, ab2, ab3], [msg0, msg1],
                  isem, gsem, ssem, (c * NS + s) * _EPW, _EPW // _CB, scale)

    plsc.subcore_barrier()
    for k in range(RPT // _CB):
        sl = pl.ds(r0 + k * _CB, _CB)
        pltpu.sync_copy(acc_sh.at[sl], aggs.at[c].at[sl])


def _agg1_sc(srcdst, alpha, h1):
    return pl.kernel(
        _agg1_body,
        out_type=jax.ShapeDtypeStruct((2, NP, D), jnp.float32),
        mesh=_mesh,
        compiler_params=pltpu.CompilerParams(needs_layout_passes=False),
        scratch_types=(
            [pltpu.VMEM((2, _CB), jnp.int32)] * 4
            + [pltpu.VMEM((H, _CB), jnp.float32)] * 4
            + [pltpu.VMEM((_CB, D), jnp.float32)] * 2
            + [pltpu.SemaphoreType.DMA] * 3
            + [pltpu.VMEM_SHARED((NP, D), jnp.float32)]
        ),
    )(srcdst, alpha, h1)


def _agg2_body(srcdst, alpha, hin, agg2, sd0, sd1, sd2, sd3, ab0, ab1, ab2,
               ab3, msg0, msg1, isem, gsem, ssem, acc_sh):
    c = lax.axis_index("c")
    s = lax.axis_index("s")
    r0 = s * RPT
    base_t = s * EPT

    def scale(msg, ab):
        @pl.loop(0, _CB, unroll=2)
        def _edge(j):
            av = plsc.load_gather(ab, [jnp.full((16,), j, jnp.int32)])
            for ch in range(8):
                sl = pl.ds(ch * 16, 16)
                msg[j, sl] = msg[j, sl] * av

    for p in range(HPC):
        _zero_acc_rows(msg0, acc_sh, r0)
        plsc.subcore_barrier()
        _agg_pipeline(srcdst, lambda ds: alpha.at[c * HPC + p].at[ds], hin,
                      acc_sh, [sd0, sd1, sd2, sd3], [ab0, ab1, ab2, ab3],
                      [msg0, msg1], isem, gsem, ssem, base_t, EPT // _CB,
                      scale)
        plsc.subcore_barrier()
        for k in range(RPT // _CB):
            sl = pl.ds(r0 + k * _CB, _CB)
            pltpu.sync_copy(acc_sh.at[sl], agg2.at[c].at[p].at[sl])


def _agg2_sc(srcdst, alpha, hin):
    return pl.kernel(
        _agg2_body,
        out_type=jax.ShapeDtypeStruct((NC, HPC, NP, D), jnp.float32),
        mesh=_mesh,
        compiler_params=pltpu.CompilerParams(needs_layout_passes=False),
        scratch_types=(
            [pltpu.VMEM((2, _CB), jnp.int32)] * 4
            + [pltpu.VMEM((_CB,), jnp.float32)] * 4
            + [pltpu.VMEM((_CB, D), jnp.float32)] * 2
            + [pltpu.SemaphoreType.DMA] * 3
            + [pltpu.VMEM_SHARED((NP, D), jnp.float32)]
        ),
    )(srcdst, alpha, hin)


def _fold_heads(a, dim):
    eye = jnp.eye(H, dtype=a.dtype)
    return (a[:, :, None] * eye[:, None, :]).reshape(H * dim, H)


def kernel(x, path_index, W1, a1_src, a1_dst, b1, W2, a2_src, a2_dst, b2):
    srcdst = jnp.pad(path_index.astype(jnp.int32), ((0, 0), (0, EP - E)),
                     constant_values=NP - 1)
    xp = jnp.pad(x, ((0, NP - N), (0, 0)))
    A1 = jnp.concatenate([_fold_heads(a1_src, 16), _fold_heads(a1_dst, 16)],
                         axis=1)
    A2 = jnp.concatenate([_fold_heads(a2_src, D), _fold_heads(a2_dst, D)],
                         axis=1)
    W2stack = W2.reshape(D, H, D).transpose(1, 0, 2)

    h1, st1 = _dense1(xp, W1, A1)
    alpha1 = _softmax_sc(st1.T, srcdst)
    aggs1 = _agg1_sc(srcdst, alpha1, h1)
    hin, st2 = _dense2(aggs1, b1, W2, A2)
    alpha2 = _softmax_sc(st2.T, srcdst)
    agg2 = _agg2_sc(srcdst, alpha2, hin)
    return _dense3(agg2, W2stack, b2)[:N]


# speedup vs baseline: 1.5488x; 1.5488x over previous
import jax
import jax.numpy as jnp
from jax import lax
from jax.experimental import pallas as pl
from jax.experimental.pallas import tpu as pltpu
from jax.experimental.pallas import tpu_sc as plsc

N = 10000
NP = 10240
E = 320000
EP = 327680
H = 8
D = 128
NC = 2
NS = 16
HPC = H // NC
EPT = EP // NS
RB = 1024
RPT = NP // NS

_mesh = plsc.VectorSubcoreMesh(core_axis_name="c", subcore_axis_name="s")


def _dense1_body(x_ref, w1_ref, a1_ref, h1_ref, st_ref):
    h1 = jnp.dot(x_ref[...], w1_ref[...], preferred_element_type=jnp.float32)
    h1_ref[...] = h1
    st_ref[...] = jnp.dot(h1, a1_ref[...], preferred_element_type=jnp.float32)


def _dense1(x, W1, A1):
    return pl.pallas_call(
        _dense1_body,
        grid=(NP // RB,),
        in_specs=[pl.BlockSpec((RB, D), lambda i: (i, 0)),
                  pl.BlockSpec((D, D), lambda i: (0, 0)),
                  pl.BlockSpec((D, 2 * H), lambda i: (0, 0))],
        out_specs=[pl.BlockSpec((RB, D), lambda i: (i, 0)),
                   pl.BlockSpec((RB, 2 * H), lambda i: (i, 0))],
        out_shape=[jax.ShapeDtypeStruct((NP, D), jnp.float32),
                   jax.ShapeDtypeStruct((NP, 2 * H), jnp.float32)],
    )(x, W1, A1)


def _dense2_body(agg_ref, b1_ref, w2_ref, a2_ref, hin_ref, st_ref):
    o = agg_ref[0] + agg_ref[1] + b1_ref[...]
    hin = jnp.where(o > 0, o, jnp.exp(o) - 1.0)
    hin_ref[...] = hin
    u2 = jnp.dot(w2_ref[...], a2_ref[...], preferred_element_type=jnp.float32)
    st_ref[...] = jnp.dot(hin, u2, preferred_element_type=jnp.float32)


def _dense2(aggs1, b1, W2, A2):
    return pl.pallas_call(
        _dense2_body,
        grid=(NP // RB,),
        in_specs=[pl.BlockSpec((2, RB, D), lambda i: (0, i, 0)),
                  pl.BlockSpec((1, D), lambda i: (0, 0)),
                  pl.BlockSpec((D, H * D), lambda i: (0, 0)),
                  pl.BlockSpec((H * D, 2 * H), lambda i: (0, 0))],
        out_specs=[pl.BlockSpec((RB, D), lambda i: (i, 0)),
                   pl.BlockSpec((RB, 2 * H), lambda i: (i, 0))],
        out_shape=[jax.ShapeDtypeStruct((NP, D), jnp.float32),
                   jax.ShapeDtypeStruct((NP, 2 * H), jnp.float32)],
    )(aggs1, b1.reshape(1, D), W2, A2)


def _dense3_body(agg_ref, w2s_ref, b2_ref, out_ref):
    acc = jnp.zeros((RB, D), jnp.float32)
    for c in range(NC):
        for p in range(HPC):
            acc = acc + jnp.dot(agg_ref[c, p], w2s_ref[c * HPC + p],
                                preferred_element_type=jnp.float32)
    out_ref[...] = acc * (1.0 / H) + b2_ref[...]


def _dense3(agg2, W2stack, b2):
    return pl.pallas_call(
        _dense3_body,
        grid=(NP // RB,),
        in_specs=[pl.BlockSpec((NC, HPC, RB, D), lambda i: (0, 0, i, 0)),
                  pl.BlockSpec((H, D, D), lambda i: (0, 0, 0)),
                  pl.BlockSpec((1, D), lambda i: (0, 0))],
        out_specs=pl.BlockSpec((RB, D), lambda i: (i, 0)),
        out_shape=jax.ShapeDtypeStruct((NP, D), jnp.float32),
    )(agg2, W2stack, b2.reshape(1, D))


_CH = 2048


def _softmax_body(stt, srcdst, alpha, s_vm, t_vm, d_vm, src_c, dst_c, ex_c,
                  den_sh):
    c = lax.axis_index("c")
    s = lax.axis_index("s")
    head = c * HPC + s // 4
    q = s % 4
    EQ = EP // 4
    base_q = q * EQ

    pltpu.sync_copy(stt.at[head], s_vm)
    pltpu.sync_copy(stt.at[H + head], t_vm)

    zero16 = jnp.zeros((16,), jnp.float32)

    @pl.loop(0, NP // 16)
    def _zero(i):
        d_vm[pl.ds(i * 16, 16)] = zero16

    @pl.loop(0, EQ // _CH)
    def _pass1(ci):
        base = base_q + ci * _CH
        pltpu.sync_copy(srcdst.at[0].at[pl.ds(base, _CH)], src_c)
        pltpu.sync_copy(srcdst.at[1].at[pl.ds(base, _CH)], dst_c)

        @pl.loop(0, _CH // 16)
        def _inner(j):
            sl = pl.ds(j * 16, 16)
            isrc = src_c[sl]
            idst = dst_c[sl]
            e = plsc.load_gather(s_vm, [isrc]) + plsc.load_gather(t_vm, [idst])
            e = jnp.where(e >= 0.0, e, 0.2 * e)
            ex = jnp.exp(e)
            ex_c[sl] = ex
            plsc.addupdate_scatter(d_vm, [idst], ex)

        pltpu.sync_copy(ex_c, alpha.at[head].at[pl.ds(base, _CH)])

    pltpu.sync_copy(d_vm, den_sh.at[s])
    plsc.subcore_barrier()
    g4 = (s // 4) * 4

    pltpu.sync_copy(den_sh.at[g4], d_vm)
    for k in range(1, 4):
        pltpu.sync_copy(den_sh.at[g4 + k], t_vm)

        @pl.loop(0, NP // 16)
        def _acc(i):
            sl = pl.ds(i * 16, 16)
            d_vm[sl] = d_vm[sl] + t_vm[sl]

    @pl.loop(0, NP // 16)
    def _recip(i):
        sl = pl.ds(i * 16, 16)
        d_vm[sl] = 1.0 / (d_vm[sl] + 1e-16)

    @pl.loop(0, EQ // _CH)
    def _pass2(ci):
        base = base_q + ci * _CH
        pltpu.sync_copy(srcdst.at[1].at[pl.ds(base, _CH)], dst_c)
        pltpu.sync_copy(alpha.at[head].at[pl.ds(base, _CH)], ex_c)

        @pl.loop(0, _CH // 16)
        def _inner(j):
            sl = pl.ds(j * 16, 16)
            rd = plsc.load_gather(d_vm, [dst_c[sl]])
            ex_c[sl] = ex_c[sl] * rd

        pltpu.sync_copy(ex_c, alpha.at[head].at[pl.ds(base, _CH)])


def _softmax_sc(stt, srcdst):
    return pl.kernel(
        _softmax_body,
        out_type=jax.ShapeDtypeStruct((H, EP), jnp.float32),
        mesh=_mesh,
        compiler_params=pltpu.CompilerParams(needs_layout_passes=False),
        scratch_types=[
            pltpu.VMEM((NP,), jnp.float32),
            pltpu.VMEM((NP,), jnp.float32),
            pltpu.VMEM((NP,), jnp.float32),
            pltpu.VMEM((_CH,), jnp.int32),
            pltpu.VMEM((_CH,), jnp.int32),
            pltpu.VMEM((_CH,), jnp.float32),
            pltpu.VMEM_SHARED((NS, NP), jnp.float32),
        ],
    )(stt, srcdst)


_CB = 128


def _agg_pipeline(srcdst, alpha_slice, feat, acc_sh, SD, AB, MSG, isem, gsem,
                  ssem, base_t, nch, scale_fn, linear_src=False):
    def issue_idx(ci, k):
        base = base_t + ci * _CB
        pltpu.async_copy(srcdst.at[:, pl.ds(base, _CB)], SD[k], isem)
        pltpu.async_copy(alpha_slice(pl.ds(base, _CB)), AB[k], isem)

    def wait_idx(k):
        pltpu.make_async_copy(srcdst.at[:, pl.ds(0, _CB)], SD[k], isem).wait()
        pltpu.make_async_copy(alpha_slice(pl.ds(0, _CB)), AB[k], isem).wait()

    def fire_gather(ci, k2, k4):
        if linear_src:
            base = base_t + ci * _CB
            pltpu.async_copy(feat.at[pl.ds(base, _CB)], MSG[k2], gsem)
        else:
            pltpu.async_copy(feat.at[SD[k4].at[0]], MSG[k2], gsem)

    def wait_gather(k2, k4):
        if linear_src:
            pltpu.make_async_copy(feat.at[pl.ds(0, _CB)], MSG[k2],
                                  gsem).wait()
        else:
            pltpu.make_async_copy(feat.at[SD[k4].at[0]], MSG[k2],
                                  gsem).wait()

    def fire_scatter(k2, k4):
        pltpu.async_copy(MSG[k2], acc_sh.at[SD[k4].at[1]], ssem, add=True)

    def wait_scatter(k2, k4):
        pltpu.make_async_copy(MSG[k2], acc_sh.at[SD[k4].at[1]], ssem).wait()

    issue_idx(0, 0)
    issue_idx(1, 1)
    wait_idx(0)
    fire_gather(0, 0, 0)

    @pl.loop(0, nch // 4)
    def _grp(g):
        for k in range(4):
            ci = g * 4 + k
            k2 = k % 2

            @pl.when(ci >= 1)
            def _():
                wait_scatter((k - 1) % 2, (k - 1) % 4)

            @pl.when(ci + 2 < nch)
            def _():
                issue_idx(ci + 2, (k + 2) % 4)

            @pl.when(ci + 1 < nch)
            def _():
                wait_idx((k + 1) % 4)
                fire_gather(ci + 1, (k + 1) % 2, (k + 1) % 4)

            wait_gather(k2, k)
            scale_fn(MSG[k2], AB[k])
            fire_scatter(k2, k)

    wait_scatter(1, 3)


def _zero_acc_rows(msg, acc_sh, r0):
    zero16 = jnp.zeros((16,), jnp.float32)

    @pl.loop(0, _CB)
    def _z(j):
        for hh in range(8):
            msg[j, pl.ds(hh * 16, 16)] = zero16

    for k in range(RPT // _CB):
        pltpu.sync_copy(msg.at[pl.ds(0, _CB)],
                        acc_sh.at[pl.ds(r0 + k * _CB, _CB)])


def _gatherg_body(srcdst, hin, g, sd0, sd1, sd2, sd3, msg0, msg1, isem, gsem,
                  ssem):
    c = lax.axis_index("c")
    s = lax.axis_index("s")
    base_t = (c * NS + s) * _EPW
    SD = [sd0, sd1, sd2, sd3]
    MSG = [msg0, msg1]
    nch = _EPW // _CB

    def issue_idx(ci, k):
        base = base_t + ci * _CB
        pltpu.async_copy(srcdst.at[0].at[pl.ds(base, _CB)], SD[k], isem)

    def wait_idx(k):
        pltpu.make_async_copy(srcdst.at[0].at[pl.ds(0, _CB)], SD[k],
                              isem).wait()

    def fire_gather(k2, k4):
        pltpu.async_copy(hin.at[SD[k4]], MSG[k2], gsem)

    def wait_gather(k2, k4):
        pltpu.make_async_copy(hin.at[SD[k4]], MSG[k2], gsem).wait()

    def fire_write(ci, k2):
        base = base_t + ci * _CB
        pltpu.async_copy(MSG[k2], g.at[pl.ds(base, _CB)], ssem)

    def wait_write(k2):
        pltpu.make_async_copy(MSG[k2], g.at[pl.ds(0, _CB)], ssem).wait()

    issue_idx(0, 0)
    issue_idx(1, 1)
    wait_idx(0)
    fire_gather(0, 0)

    @pl.loop(0, nch // 4)
    def _grp(g_):
        for k in range(4):
            ci = g_ * 4 + k
            k2 = k % 2

            @pl.when(ci >= 1)
            def _():
                wait_write((k - 1) % 2)

            @pl.when(ci + 2 < nch)
            def _():
                issue_idx(ci + 2, (k + 2) % 4)

            @pl.when(ci + 1 < nch)
            def _():
                wait_idx((k + 1) % 4)
                fire_gather((k + 1) % 2, (k + 1) % 4)

            wait_gather(k2, k)
            fire_write(ci, k2)

    wait_write(1)


def _gatherg_sc(srcdst, hin):
    return pl.kernel(
        _gatherg_body,
        out_type=jax.ShapeDtypeStruct((EP, D), jnp.float32),
        mesh=_mesh,
        compiler_params=pltpu.CompilerParams(needs_layout_passes=False),
        scratch_types=(
            [pltpu.VMEM((_CB,), jnp.int32)] * 4
            + [pltpu.VMEM((_CB, D), jnp.float32)] * 2
            + [pltpu.SemaphoreType.DMA] * 3
        ),
    )(srcdst, hin)


_EPW = EP // (NC * NS)


def _agg1_body(srcdst, alpha, h1, aggs, sd0, sd1, sd2, sd3, ab0, ab1, ab2,
               ab3, msg0, msg1, isem, gsem, ssem, acc_sh):
    c = lax.axis_index("c")
    s = lax.axis_index("s")
    r0 = s * RPT
    _zero_acc_rows(msg0, acc_sh, r0)
    plsc.subcore_barrier()

    def scale(msg, ab):
        @pl.loop(0, _CB, unroll=2)
        def _edge(j):
            jj = jnp.full((16,), j, jnp.int32)
            for hh in range(H):
                av = plsc.load_gather(ab, [jnp.full((16,), hh, jnp.int32), jj])
                sl = pl.ds(hh * 16, 16)
                msg[j, sl] = msg[j, sl] * av

    _agg_pipeline(srcdst, lambda ds: alpha.at[:, ds], h1, acc_sh,
                  [sd0, sd1, sd2, sd3], [ab0, ab1, ab2, ab3], [msg0, msg1],
                  isem, gsem, ssem, (c * NS + s) * _EPW, _EPW // _CB, scale)

    plsc.subcore_barrier()
    for k in range(RPT // _CB):
        sl = pl.ds(r0 + k * _CB, _CB)
        pltpu.sync_copy(acc_sh.at[sl], aggs.at[c].at[sl])


def _agg1_sc(srcdst, alpha, h1):
    return pl.kernel(
        _agg1_body,
        out_type=jax.ShapeDtypeStruct((2, NP, D), jnp.float32),
        mesh=_mesh,
        compiler_params=pltpu.CompilerParams(needs_layout_passes=False),
        scratch_types=(
            [pltpu.VMEM((2, _CB), jnp.int32)] * 4
            + [pltpu.VMEM((H, _CB), jnp.float32)] * 4
            + [pltpu.VMEM((_CB, D), jnp.float32)] * 2
            + [pltpu.SemaphoreType.DMA] * 3
            + [pltpu.VMEM_SHARED((NP, D), jnp.float32)]
        ),
    )(srcdst, alpha, h1)


def _agg2_body(srcdst, alpha, g, agg2, sd0, sd1, sd2, sd3, ab0, ab1, ab2,
               ab3, msg0, msg1, isem, gsem, ssem, acc_sh):
    c = lax.axis_index("c")
    s = lax.axis_index("s")
    r0 = s * RPT
    base_t = s * EPT

    def scale(msg, ab):
        @pl.loop(0, _CB, unroll=2)
        def _edge(j):
            av = plsc.load_gather(ab, [jnp.full((16,), j, jnp.int32)])
            for ch in range(8):
                sl = pl.ds(ch * 16, 16)
                msg[j, sl] = msg[j, sl] * av

    for p in range(HPC):
        _zero_acc_rows(msg0, acc_sh, r0)
        plsc.subcore_barrier()
        _agg_pipeline(srcdst, lambda ds: alpha.at[c * HPC + p].at[ds], g,
                      acc_sh, [sd0, sd1, sd2, sd3], [ab0, ab1, ab2, ab3],
                      [msg0, msg1], isem, gsem, ssem, base_t, EPT // _CB,
                      scale, linear_src=True)
        plsc.subcore_barrier()
        for k in range(RPT // _CB):
            sl = pl.ds(r0 + k * _CB, _CB)
            pltpu.sync_copy(acc_sh.at[sl], agg2.at[c].at[p].at[sl])


def _agg2_sc(srcdst, alpha, g):
    return pl.kernel(
        _agg2_body,
        out_type=jax.ShapeDtypeStruct((NC, HPC, NP, D), jnp.float32),
        mesh=_mesh,
        compiler_params=pltpu.CompilerParams(needs_layout_passes=False),
        scratch_types=(
            [pltpu.VMEM((2, _CB), jnp.int32)] * 4
            + [pltpu.VMEM((_CB,), jnp.float32)] * 4
            + [pltpu.VMEM((_CB, D), jnp.float32)] * 2
            + [pltpu.SemaphoreType.DMA] * 3
            + [pltpu.VMEM_SHARED((NP, D), jnp.float32)]
        ),
    )(srcdst, alpha, g)


def _fold_heads(a, dim):
    eye = jnp.eye(H, dtype=a.dtype)
    return (a[:, :, None] * eye[:, None, :]).reshape(H * dim, H)


def kernel(x, path_index, W1, a1_src, a1_dst, b1, W2, a2_src, a2_dst, b2):
    srcdst = jnp.pad(path_index.astype(jnp.int32), ((0, 0), (0, EP - E)),
                     constant_values=NP - 1)
    xp = jnp.pad(x, ((0, NP - N), (0, 0)))
    A1 = jnp.concatenate([_fold_heads(a1_src, 16), _fold_heads(a1_dst, 16)],
                         axis=1)
    A2 = jnp.concatenate([_fold_heads(a2_src, D), _fold_heads(a2_dst, D)],
                         axis=1)
    W2stack = W2.reshape(D, H, D).transpose(1, 0, 2)

    h1, st1 = _dense1(xp, W1, A1)
    alpha1 = _softmax_sc(st1.T, srcdst)
    aggs1 = _agg1_sc(srcdst, alpha1, h1)
    hin, st2 = _dense2(aggs1, b1, W2, A2)
    g = _gatherg_sc(srcdst, hin)
    alpha2 = _softmax_sc(st2.T, srcdst)
    agg2 = _agg2_sc(srcdst, alpha2, g)
    return _dense3(agg2, W2stack, b2)[:N]


# speedup vs baseline: 1.6432x; 1.0610x over previous
import jax
import jax.numpy as jnp
from jax import lax
from jax.experimental import pallas as pl
from jax.experimental.pallas import tpu as pltpu
from jax.experimental.pallas import tpu_sc as plsc

N = 10000
NP = 10240
E = 320000
EP = 327680
H = 8
D = 128
NC = 2
NS = 16
HPC = H // NC
EPT = EP // NS
RB = 1024
RPT = NP // NS

_mesh = plsc.VectorSubcoreMesh(core_axis_name="c", subcore_axis_name="s")


def _dense1_body(x_ref, w1_ref, a1_ref, h1_ref, st_ref):
    h1 = jnp.dot(x_ref[...], w1_ref[...], preferred_element_type=jnp.float32)
    h1_ref[...] = h1
    st_ref[...] = jnp.dot(h1, a1_ref[...], preferred_element_type=jnp.float32)


def _dense1(x, W1, A1):
    return pl.pallas_call(
        _dense1_body,
        grid=(NP // RB,),
        in_specs=[pl.BlockSpec((RB, D), lambda i: (i, 0)),
                  pl.BlockSpec((D, D), lambda i: (0, 0)),
                  pl.BlockSpec((D, 2 * H), lambda i: (0, 0))],
        out_specs=[pl.BlockSpec((RB, D), lambda i: (i, 0)),
                   pl.BlockSpec((RB, 2 * H), lambda i: (i, 0))],
        out_shape=[jax.ShapeDtypeStruct((NP, D), jnp.float32),
                   jax.ShapeDtypeStruct((NP, 2 * H), jnp.float32)],
    )(x, W1, A1)


def _dense2_body(agg_ref, b1_ref, w2_ref, a2_ref, hin_ref, st_ref):
    o = agg_ref[0] + agg_ref[1] + b1_ref[...]
    hin = jnp.where(o > 0, o, jnp.exp(o) - 1.0)
    hin_ref[...] = hin
    u2 = jnp.dot(w2_ref[...], a2_ref[...], preferred_element_type=jnp.float32)
    st_ref[...] = jnp.dot(hin, u2, preferred_element_type=jnp.float32)


def _dense2(aggs1, b1, W2, A2):
    return pl.pallas_call(
        _dense2_body,
        grid=(NP // RB,),
        in_specs=[pl.BlockSpec((2, RB, D), lambda i: (0, i, 0)),
                  pl.BlockSpec((1, D), lambda i: (0, 0)),
                  pl.BlockSpec((D, H * D), lambda i: (0, 0)),
                  pl.BlockSpec((H * D, 2 * H), lambda i: (0, 0))],
        out_specs=[pl.BlockSpec((RB, D), lambda i: (i, 0)),
                   pl.BlockSpec((RB, 2 * H), lambda i: (i, 0))],
        out_shape=[jax.ShapeDtypeStruct((NP, D), jnp.float32),
                   jax.ShapeDtypeStruct((NP, 2 * H), jnp.float32)],
    )(aggs1, b1.reshape(1, D), W2, A2)


def _dense3_body(agg_ref, w2s_ref, b2_ref, out_ref):
    acc = jnp.zeros((RB, D), jnp.float32)
    for c in range(NC):
        for p in range(HPC):
            acc = acc + jnp.dot(agg_ref[c, p], w2s_ref[c * HPC + p],
                                preferred_element_type=jnp.float32)
    out_ref[...] = acc * (1.0 / H) + b2_ref[...]


def _dense3(agg2, W2stack, b2):
    return pl.pallas_call(
        _dense3_body,
        grid=(NP // RB,),
        in_specs=[pl.BlockSpec((NC, HPC, RB, D), lambda i: (0, 0, i, 0)),
                  pl.BlockSpec((H, D, D), lambda i: (0, 0, 0)),
                  pl.BlockSpec((1, D), lambda i: (0, 0))],
        out_specs=pl.BlockSpec((RB, D), lambda i: (i, 0)),
        out_shape=jax.ShapeDtypeStruct((NP, D), jnp.float32),
    )(agg2, W2stack, b2.reshape(1, D))


_CH = 2048


def _softmax_body(stt, srcdst, alpha, s_vm, t_vm, d_vm,
                  sd0, sd1, sd2, sd3, ex0, ex1, ex2, ex3, isem, wsem,
                  den_sh):
    c = lax.axis_index("c")
    s = lax.axis_index("s")
    head = c * HPC + s // 4
    q = s % 4
    EQ = EP // 4
    base_q = q * EQ
    nch = EQ // _CH
    SD = [sd0, sd1, sd2, sd3]
    EX = [ex0, ex1, ex2, ex3]

    pltpu.sync_copy(stt.at[head], s_vm)
    pltpu.sync_copy(stt.at[H + head], t_vm)

    zero16 = jnp.zeros((16,), jnp.float32)

    @pl.loop(0, NP // 16)
    def _zero(i):
        d_vm[pl.ds(i * 16, 16)] = zero16

    def issue1(ci, k):
        base = base_q + ci * _CH
        pltpu.async_copy(srcdst.at[:, pl.ds(base, _CH)], SD[k], isem)

    def wait1(k):
        pltpu.make_async_copy(srcdst.at[:, pl.ds(0, _CH)], SD[k],
                              isem).wait()

    def fire_wr(ci, k):
        base = base_q + ci * _CH
        pltpu.async_copy(EX[k], alpha.at[head].at[pl.ds(base, _CH)], wsem)

    def wait_wr(k):
        pltpu.make_async_copy(EX[k], alpha.at[head].at[pl.ds(0, _CH)],
                              wsem).wait()

    issue1(0, 0)
    issue1(1, 1)

    @pl.loop(0, nch // 4)
    def _p1(g):
        for k in range(4):
            ci = g * 4 + k

            @pl.when(ci >= 1)
            def _():
                wait_wr((k - 1) % 4)

            @pl.when(ci + 2 < nch)
            def _():
                issue1(ci + 2, (k + 2) % 4)

            wait1(k)

            @pl.loop(0, _CH // 16, unroll=2)
            def _inner(j):
                sl = pl.ds(j * 16, 16)
                isrc = SD[k][0, sl]
                idst = SD[k][1, sl]
                e = (plsc.load_gather(s_vm, [isrc])
                     + plsc.load_gather(t_vm, [idst]))
                e = jnp.where(e >= 0.0, e, 0.2 * e)
                ex = jnp.exp(e)
                EX[k][sl] = ex
                plsc.addupdate_scatter(d_vm, [idst], ex)

            fire_wr(ci, k)

    wait_wr(3)

    pltpu.sync_copy(d_vm, den_sh.at[s])
    plsc.subcore_barrier()
    g4 = (s // 4) * 4

    pltpu.sync_copy(den_sh.at[g4], d_vm)
    for k in range(1, 4):
        pltpu.sync_copy(den_sh.at[g4 + k], t_vm)

        @pl.loop(0, NP // 16)
        def _acc(i):
            sl = pl.ds(i * 16, 16)
            d_vm[sl] = d_vm[sl] + t_vm[sl]

    @pl.loop(0, NP // 16)
    def _recip(i):
        sl = pl.ds(i * 16, 16)
        d_vm[sl] = 1.0 / (d_vm[sl] + 1e-16)

    def issue2(ci, k):
        base = base_q + ci * _CH
        pltpu.async_copy(srcdst.at[:, pl.ds(base, _CH)], SD[k], isem)
        pltpu.async_copy(alpha.at[head].at[pl.ds(base, _CH)], EX[k], isem)

    def wait2(k):
        pltpu.make_async_copy(srcdst.at[:, pl.ds(0, _CH)], SD[k],
                              isem).wait()
        pltpu.make_async_copy(alpha.at[head].at[pl.ds(0, _CH)], EX[k],
                              isem).wait()

    issue2(0, 0)
    issue2(1, 1)

    @pl.loop(0, nch // 4)
    def _p2(g):
        for k in range(4):
            ci = g * 4 + k

            @pl.when(ci >= 1)
            def _():
                wait_wr((k - 1) % 4)

            @pl.when(ci + 2 < nch)
            def _():
                issue2(ci + 2, (k + 2) % 4)

            wait2(k)

            @pl.loop(0, _CH // 16, unroll=2)
            def _inner(j):
                sl = pl.ds(j * 16, 16)
                rd = plsc.load_gather(d_vm, [SD[k][1, sl]])
                EX[k][sl] = EX[k][sl] * rd

            fire_wr(ci, k)

    wait_wr(3)


def _softmax_sc(stt, srcdst):
    return pl.kernel(
        _softmax_body,
        out_type=jax.ShapeDtypeStruct((H, EP), jnp.float32),
        mesh=_mesh,
        compiler_params=pltpu.CompilerParams(needs_layout_passes=False),
        scratch_types=(
            [pltpu.VMEM((NP,), jnp.float32)] * 3
            + [pltpu.VMEM((2, _CH), jnp.int32)] * 4
            + [pltpu.VMEM((_CH,), jnp.float32)] * 4
            + [pltpu.SemaphoreType.DMA] * 2
            + [pltpu.VMEM_SHARED((NS, NP), jnp.float32)]
        ),
    )(stt, srcdst)


_CB = 128


def _agg_pipeline(srcdst, alpha_slice, feat, acc_sh, SD, AB, MSG, isem, gsem,
                  ssem, base_t, nch, scale_fn, linear_src=False):
    def issue_idx(ci, k):
        base = base_t + ci * _CB
        pltpu.async_copy(srcdst.at[:, pl.ds(base, _CB)], SD[k], isem)
        pltpu.async_copy(alpha_slice(pl.ds(base, _CB)), AB[k], isem)

    def wait_idx(k):
        pltpu.make_async_copy(srcdst.at[:, pl.ds(0, _CB)], SD[k], isem).wait()
        pltpu.make_async_copy(alpha_slice(pl.ds(0, _CB)), AB[k], isem).wait()

    def fire_gather(ci, k2, k4):
        if linear_src:
            base = base_t + ci * _CB
            pltpu.async_copy(feat.at[pl.ds(base, _CB)], MSG[k2], gsem)
        else:
            pltpu.async_copy(feat.at[SD[k4].at[0]], MSG[k2], gsem)

    def wait_gather(k2, k4):
        if linear_src:
            pltpu.make_async_copy(feat.at[pl.ds(0, _CB)], MSG[k2],
                                  gsem).wait()
        else:
            pltpu.make_async_copy(feat.at[SD[k4].at[0]], MSG[k2],
                                  gsem).wait()

    def fire_scatter(k2, k4):
        pltpu.async_copy(MSG[k2], acc_sh.at[SD[k4].at[1]], ssem, add=True)

    def wait_scatter(k2, k4):
        pltpu.make_async_copy(MSG[k2], acc_sh.at[SD[k4].at[1]], ssem).wait()

    issue_idx(0, 0)
    issue_idx(1, 1)
    wait_idx(0)
    fire_gather(0, 0, 0)

    @pl.loop(0, nch // 4)
    def _grp(g):
        for k in range(4):
            ci = g * 4 + k
            k2 = k % 2

            @pl.when(ci >= 1)
            def _():
                wait_scatter((k - 1) % 2, (k - 1) % 4)

            @pl.when(ci + 2 < nch)
            def _():
                issue_idx(ci + 2, (k + 2) % 4)

            @pl.when(ci + 1 < nch)
            def _():
                wait_idx((k + 1) % 4)
                fire_gather(ci + 1, (k + 1) % 2, (k + 1) % 4)

            wait_gather(k2, k)
            scale_fn(MSG[k2], AB[k])
            fire_scatter(k2, k)

    wait_scatter(1, 3)


def _zero_acc_rows(msg, acc_sh, r0):
    zero16 = jnp.zeros((16,), jnp.float32)

    @pl.loop(0, _CB)
    def _z(j):
        for hh in range(8):
            msg[j, pl.ds(hh * 16, 16)] = zero16

    for k in range(RPT // _CB):
        pltpu.sync_copy(msg.at[pl.ds(0, _CB)],
                        acc_sh.at[pl.ds(r0 + k * _CB, _CB)])


def _gatherg_body(srcdst, hin, g, sd0, sd1, sd2, sd3, msg0, msg1, isem, gsem,
                  ssem):
    c = lax.axis_index("c")
    s = lax.axis_index("s")
    base_t = (c * NS + s) * _EPW
    SD = [sd0, sd1, sd2, sd3]
    MSG = [msg0, msg1]
    nch = _EPW // _CB

    def issue_idx(ci, k):
        base = base_t + ci * _CB
        pltpu.async_copy(srcdst.at[0].at[pl.ds(base, _CB)], SD[k], isem)

    def wait_idx(k):
        pltpu.make_async_copy(srcdst.at[0].at[pl.ds(0, _CB)], SD[k],
                              isem).wait()

    def fire_gather(k2, k4):
        pltpu.async_copy(hin.at[SD[k4]], MSG[k2], gsem)

    def wait_gather(k2, k4):
        pltpu.make_async_copy(hin.at[SD[k4]], MSG[k2], gsem).wait()

    def fire_write(ci, k2):
        base = base_t + ci * _CB
        pltpu.async_copy(MSG[k2], g.at[pl.ds(base, _CB)], ssem)

    def wait_write(k2):
        pltpu.make_async_copy(MSG[k2], g.at[pl.ds(0, _CB)], ssem).wait()

    issue_idx(0, 0)
    issue_idx(1, 1)
    wait_idx(0)
    fire_gather(0, 0)

    @pl.loop(0, nch // 4)
    def _grp(g_):
        for k in range(4):
            ci = g_ * 4 + k
            k2 = k % 2

            @pl.when(ci >= 1)
            def _():
                wait_write((k - 1) % 2)

            @pl.when(ci + 2 < nch)
            def _():
                issue_idx(ci + 2, (k + 2) % 4)

            @pl.when(ci + 1 < nch)
            def _():
                wait_idx((k + 1) % 4)
                fire_gather((k + 1) % 2, (k + 1) % 4)

            wait_gather(k2, k)
            fire_write(ci, k2)

    wait_write(1)


def _gatherg_sc(srcdst, hin):
    return pl.kernel(
        _gatherg_body,
        out_type=jax.ShapeDtypeStruct((EP, D), jnp.float32),
        mesh=_mesh,
        compiler_params=pltpu.CompilerParams(needs_layout_passes=False),
        scratch_types=(
            [pltpu.VMEM((_CB,), jnp.int32)] * 4
            + [pltpu.VMEM((_CB, D), jnp.float32)] * 2
            + [pltpu.SemaphoreType.DMA] * 3
        ),
    )(srcdst, hin)


_EPW = EP // (NC * NS)


def _agg1_body(srcdst, alpha, h1, aggs, sd0, sd1, sd2, sd3, ab0, ab1, ab2,
               ab3, msg0, msg1, isem, gsem, ssem, acc_sh):
    c = lax.axis_index("c")
    s = lax.axis_index("s")
    r0 = s * RPT
    _zero_acc_rows(msg0, acc_sh, r0)
    plsc.subcore_barrier()

    def scale(msg, ab):
        @pl.loop(0, _CB, unroll=2)
        def _edge(j):
            jj = jnp.full((16,), j, jnp.int32)
            for hh in range(H):
                av = plsc.load_gather(ab, [jnp.full((16,), hh, jnp.int32), jj])
                sl = pl.ds(hh * 16, 16)
                msg[j, sl] = msg[j, sl] * av

    _agg_pipeline(srcdst, lambda ds: alpha.at[:, ds], h1, acc_sh,
                  [sd0, sd1, sd2, sd3], [ab0, ab1, ab2, ab3], [msg0, msg1],
                  isem, gsem, ssem, (c * NS + s) * _EPW, _EPW // _CB, scale)

    plsc.subcore_barrier()
    for k in range(RPT // _CB):
        sl = pl.ds(r0 + k * _CB, _CB)
        pltpu.sync_copy(acc_sh.at[sl], aggs.at[c].at[sl])


def _agg1_sc(srcdst, alpha, h1):
    return pl.kernel(
        _agg1_body,
        out_type=jax.ShapeDtypeStruct((2, NP, D), jnp.float32),
        mesh=_mesh,
        compiler_params=pltpu.CompilerParams(needs_layout_passes=False),
        scratch_types=(
            [pltpu.VMEM((2, _CB), jnp.int32)] * 4
            + [pltpu.VMEM((H, _CB), jnp.float32)] * 4
            + [pltpu.VMEM((_CB, D), jnp.float32)] * 2
            + [pltpu.SemaphoreType.DMA] * 3
            + [pltpu.VMEM_SHARED((NP, D), jnp.float32)]
        ),
    )(srcdst, alpha, h1)


def _agg2_body(srcdst, alpha, g, agg2, sd0, sd1, sd2, sd3, ab0, ab1, ab2,
               ab3, msg0, msg1, isem, gsem, ssem, acc_sh):
    c = lax.axis_index("c")
    s = lax.axis_index("s")
    r0 = s * RPT
    base_t = s * EPT

    def scale(msg, ab):
        @pl.loop(0, _CB, unroll=2)
        def _edge(j):
            av = plsc.load_gather(ab, [jnp.full((16,), j, jnp.int32)])
            for ch in range(8):
                sl = pl.ds(ch * 16, 16)
                msg[j, sl] = msg[j, sl] * av

    for p in range(HPC):
        _zero_acc_rows(msg0, acc_sh, r0)
        plsc.subcore_barrier()
        _agg_pipeline(srcdst, lambda ds: alpha.at[c * HPC + p].at[ds], g,
                      acc_sh, [sd0, sd1, sd2, sd3], [ab0, ab1, ab2, ab3],
                      [msg0, msg1], isem, gsem, ssem, base_t, EPT // _CB,
                      scale, linear_src=True)
        plsc.subcore_barrier()
        for k in range(RPT // _CB):
            sl = pl.ds(r0 + k * _CB, _CB)
            pltpu.sync_copy(acc_sh.at[sl], agg2.at[c].at[p].at[sl])


def _agg2_sc(srcdst, alpha, g):
    return pl.kernel(
        _agg2_body,
        out_type=jax.ShapeDtypeStruct((NC, HPC, NP, D), jnp.float32),
        mesh=_mesh,
        compiler_params=pltpu.CompilerParams(needs_layout_passes=False),
        scratch_types=(
            [pltpu.VMEM((2, _CB), jnp.int32)] * 4
            + [pltpu.VMEM((_CB,), jnp.float32)] * 4
            + [pltpu.VMEM((_CB, D), jnp.float32)] * 2
            + [pltpu.SemaphoreType.DMA] * 3
            + [pltpu.VMEM_SHARED((NP, D), jnp.float32)]
        ),
    )(srcdst, alpha, g)


def _fold_heads(a, dim):
    eye = jnp.eye(H, dtype=a.dtype)
    return (a[:, :, None] * eye[:, None, :]).reshape(H * dim, H)


def kernel(x, path_index, W1, a1_src, a1_dst, b1, W2, a2_src, a2_dst, b2):
    srcdst = jnp.pad(path_index.astype(jnp.int32), ((0, 0), (0, EP - E)),
                     constant_values=NP - 1)
    xp = jnp.pad(x, ((0, NP - N), (0, 0)))
    A1 = jnp.concatenate([_fold_heads(a1_src, 16), _fold_heads(a1_dst, 16)],
                         axis=1)
    A2 = jnp.concatenate([_fold_heads(a2_src, D), _fold_heads(a2_dst, D)],
                         axis=1)
    W2stack = W2.reshape(D, H, D).transpose(1, 0, 2)

    h1, st1 = _dense1(xp, W1, A1)
    alpha1 = _softmax_sc(st1.T, srcdst)
    aggs1 = _agg1_sc(srcdst, alpha1, h1)
    hin, st2 = _dense2(aggs1, b1, W2, A2)
    g = _gatherg_sc(srcdst, hin)
    alpha2 = _softmax_sc(st2.T, srcdst)
    agg2 = _agg2_sc(srcdst, alpha2, g)
    return _dense3(agg2, W2stack, b2)[:N]


# speedup vs baseline: 1.8333x; 1.1157x over previous
import jax
import jax.numpy as jnp
from jax import lax
from jax.experimental import pallas as pl
from jax.experimental.pallas import tpu as pltpu
from jax.experimental.pallas import tpu_sc as plsc

N = 10000
NP = 10240
E = 320000
EP = 327680
H = 8
D = 128
NC = 2
NS = 16
HPC = H // NC
EPT = EP // NS
RB = 1024
RPT = NP // NS

_mesh = plsc.VectorSubcoreMesh(core_axis_name="c", subcore_axis_name="s")


def _dense1_body(x_ref, w1_ref, a1_ref, h1_ref, st_ref):
    h1 = jnp.dot(x_ref[...], w1_ref[...], preferred_element_type=jnp.float32)
    h1_ref[...] = h1
    st_ref[...] = jnp.dot(h1, a1_ref[...], preferred_element_type=jnp.float32)


def _dense1(x, W1, A1):
    return pl.pallas_call(
        _dense1_body,
        grid=(NP // RB,),
        in_specs=[pl.BlockSpec((RB, D), lambda i: (i, 0)),
                  pl.BlockSpec((D, D), lambda i: (0, 0)),
                  pl.BlockSpec((D, 2 * H), lambda i: (0, 0))],
        out_specs=[pl.BlockSpec((RB, D), lambda i: (i, 0)),
                   pl.BlockSpec((RB, 2 * H), lambda i: (i, 0))],
        out_shape=[jax.ShapeDtypeStruct((NP, D), jnp.float32),
                   jax.ShapeDtypeStruct((NP, 2 * H), jnp.float32)],
    )(x, W1, A1)


def _dense2_body(agg_ref, b1_ref, w2_ref, a2_ref, hin_ref, st_ref):
    o = agg_ref[0] + agg_ref[1] + b1_ref[...]
    hin = jnp.where(o > 0, o, jnp.exp(o) - 1.0)
    hin_ref[...] = hin
    u2 = jnp.dot(w2_ref[...], a2_ref[...], preferred_element_type=jnp.float32)
    st_ref[...] = jnp.dot(hin, u2, preferred_element_type=jnp.float32)


def _dense2(aggs1, b1, W2, A2):
    return pl.pallas_call(
        _dense2_body,
        grid=(NP // RB,),
        in_specs=[pl.BlockSpec((2, RB, D), lambda i: (0, i, 0)),
                  pl.BlockSpec((1, D), lambda i: (0, 0)),
                  pl.BlockSpec((D, H * D), lambda i: (0, 0)),
                  pl.BlockSpec((H * D, 2 * H), lambda i: (0, 0))],
        out_specs=[pl.BlockSpec((RB, D), lambda i: (i, 0)),
                   pl.BlockSpec((RB, 2 * H), lambda i: (i, 0))],
        out_shape=[jax.ShapeDtypeStruct((NP, D), jnp.float32),
                   jax.ShapeDtypeStruct((NP, 2 * H), jnp.float32)],
    )(aggs1, b1.reshape(1, D), W2, A2)


def _dense3_body(agg_ref, w2s_ref, b2_ref, out_ref):
    acc = jnp.zeros((RB, D), jnp.float32)
    for c in range(NC):
        for p in range(HPC):
            acc = acc + jnp.dot(agg_ref[c, p], w2s_ref[c * HPC + p],
                                preferred_element_type=jnp.float32)
    out_ref[...] = acc * (1.0 / H) + b2_ref[...]


def _dense3(agg2, W2stack, b2):
    return pl.pallas_call(
        _dense3_body,
        grid=(NP // RB,),
        in_specs=[pl.BlockSpec((NC, HPC, RB, D), lambda i: (0, 0, i, 0)),
                  pl.BlockSpec((H, D, D), lambda i: (0, 0, 0)),
                  pl.BlockSpec((1, D), lambda i: (0, 0))],
        out_specs=pl.BlockSpec((RB, D), lambda i: (i, 0)),
        out_shape=jax.ShapeDtypeStruct((NP, D), jnp.float32),
    )(agg2, W2stack, b2.reshape(1, D))


_CH = 2048


def _softmax_body(stt, srcdst, alpha, s_vm, t_vm, d_vm,
                  sd0, sd1, sd2, sd3, ex0, ex1, ex2, ex3, isem, wsem,
                  den_sh):
    c = lax.axis_index("c")
    s = lax.axis_index("s")
    head = c * HPC + s // 4
    q = s % 4
    EQ = EP // 4
    base_q = q * EQ
    nch = EQ // _CH
    SD = [sd0, sd1, sd2, sd3]
    EX = [ex0, ex1, ex2, ex3]

    pltpu.sync_copy(stt.at[head], s_vm)
    pltpu.sync_copy(stt.at[H + head], t_vm)

    zero16 = jnp.zeros((16,), jnp.float32)

    @pl.loop(0, NP // 16)
    def _zero(i):
        d_vm[pl.ds(i * 16, 16)] = zero16

    def issue1(ci, k):
        base = base_q + ci * _CH
        pltpu.async_copy(srcdst.at[:, pl.ds(base, _CH)], SD[k], isem)

    def wait1(k):
        pltpu.make_async_copy(srcdst.at[:, pl.ds(0, _CH)], SD[k],
                              isem).wait()

    def fire_wr(ci, k):
        base = base_q + ci * _CH
        pltpu.async_copy(EX[k], alpha.at[head].at[pl.ds(base, _CH)], wsem)

    def wait_wr(k):
        pltpu.make_async_copy(EX[k], alpha.at[head].at[pl.ds(0, _CH)],
                              wsem).wait()

    issue1(0, 0)
    issue1(1, 1)

    @pl.loop(0, nch // 4)
    def _p1(g):
        for k in range(4):
            ci = g * 4 + k

            @pl.when(ci >= 1)
            def _():
                wait_wr((k - 1) % 4)

            @pl.when(ci + 2 < nch)
            def _():
                issue1(ci + 2, (k + 2) % 4)

            wait1(k)

            @pl.loop(0, _CH // 16, unroll=2)
            def _inner(j):
                sl = pl.ds(j * 16, 16)
                isrc = SD[k][0, sl]
                idst = SD[k][1, sl]
                e = (plsc.load_gather(s_vm, [isrc])
                     + plsc.load_gather(t_vm, [idst]))
                e = jnp.where(e >= 0.0, e, 0.2 * e)
                ex = jnp.exp(e)
                EX[k][sl] = ex
                plsc.addupdate_scatter(d_vm, [idst], ex)

            fire_wr(ci, k)

    wait_wr(3)

    pltpu.sync_copy(d_vm, den_sh.at[s])
    plsc.subcore_barrier()
    g4 = (s // 4) * 4

    pltpu.sync_copy(den_sh.at[g4], d_vm)
    for k in range(1, 4):
        pltpu.sync_copy(den_sh.at[g4 + k], t_vm)

        @pl.loop(0, NP // 16)
        def _acc(i):
            sl = pl.ds(i * 16, 16)
            d_vm[sl] = d_vm[sl] + t_vm[sl]

    @pl.loop(0, NP // 16)
    def _recip(i):
        sl = pl.ds(i * 16, 16)
        d_vm[sl] = 1.0 / (d_vm[sl] + 1e-16)

    def issue2(ci, k):
        base = base_q + ci * _CH
        pltpu.async_copy(srcdst.at[:, pl.ds(base, _CH)], SD[k], isem)
        pltpu.async_copy(alpha.at[head].at[pl.ds(base, _CH)], EX[k], isem)

    def wait2(k):
        pltpu.make_async_copy(srcdst.at[:, pl.ds(0, _CH)], SD[k],
                              isem).wait()
        pltpu.make_async_copy(alpha.at[head].at[pl.ds(0, _CH)], EX[k],
                              isem).wait()

    issue2(0, 0)
    issue2(1, 1)

    @pl.loop(0, nch // 4)
    def _p2(g):
        for k in range(4):
            ci = g * 4 + k

            @pl.when(ci >= 1)
            def _():
                wait_wr((k - 1) % 4)

            @pl.when(ci + 2 < nch)
            def _():
                issue2(ci + 2, (k + 2) % 4)

            wait2(k)

            @pl.loop(0, _CH // 16, unroll=2)
            def _inner(j):
                sl = pl.ds(j * 16, 16)
                rd = plsc.load_gather(d_vm, [SD[k][1, sl]])
                EX[k][sl] = EX[k][sl] * rd

            fire_wr(ci, k)

    wait_wr(3)


def _softmax_sc(stt, srcdst):
    return pl.kernel(
        _softmax_body,
        out_type=jax.ShapeDtypeStruct((H, EP), jnp.float32),
        mesh=_mesh,
        compiler_params=pltpu.CompilerParams(needs_layout_passes=False),
        scratch_types=(
            [pltpu.VMEM((NP,), jnp.float32)] * 3
            + [pltpu.VMEM((2, _CH), jnp.int32)] * 4
            + [pltpu.VMEM((_CH,), jnp.float32)] * 4
            + [pltpu.SemaphoreType.DMA] * 2
            + [pltpu.VMEM_SHARED((NS, NP), jnp.float32)]
        ),
    )(stt, srcdst)


_CB = 128


def _agg_pipeline(srcdst, alpha_slice, feat, acc_sh, SD, AB, MSG, isem, gsem,
                  ssem, base_t, nch, scale_fn, linear_src=False):
    def issue_idx(ci, k):
        base = base_t + ci * _CB
        pltpu.async_copy(srcdst.at[:, pl.ds(base, _CB)], SD[k], isem)
        pltpu.async_copy(alpha_slice(pl.ds(base, _CB)), AB[k], isem)

    def wait_idx(k):
        pltpu.make_async_copy(srcdst.at[:, pl.ds(0, _CB)], SD[k], isem).wait()
        pltpu.make_async_copy(alpha_slice(pl.ds(0, _CB)), AB[k], isem).wait()

    def fire_gather(ci, k2, k4):
        if linear_src:
            base = base_t + ci * _CB
            pltpu.async_copy(feat.at[pl.ds(base, _CB)], MSG[k2], gsem)
        else:
            pltpu.async_copy(feat.at[SD[k4].at[0]], MSG[k2], gsem)

    def wait_gather(k2, k4):
        if linear_src:
            pltpu.make_async_copy(feat.at[pl.ds(0, _CB)], MSG[k2],
                                  gsem).wait()
        else:
            pltpu.make_async_copy(feat.at[SD[k4].at[0]], MSG[k2],
                                  gsem).wait()

    def fire_scatter(k2, k4):
        pltpu.async_copy(MSG[k2], acc_sh.at[SD[k4].at[1]], ssem, add=True)

    def wait_scatter(k2, k4):
        pltpu.make_async_copy(MSG[k2], acc_sh.at[SD[k4].at[1]], ssem).wait()

    issue_idx(0, 0)
    issue_idx(1, 1)
    wait_idx(0)
    fire_gather(0, 0, 0)

    @pl.loop(0, nch // 4)
    def _grp(g):
        for k in range(4):
            ci = g * 4 + k
            k2 = k % 2

            @pl.when(ci >= 1)
            def _():
                wait_scatter((k - 1) % 2, (k - 1) % 4)

            @pl.when(ci + 2 < nch)
            def _():
                issue_idx(ci + 2, (k + 2) % 4)

            @pl.when(ci + 1 < nch)
            def _():
                wait_idx((k + 1) % 4)
                fire_gather(ci + 1, (k + 1) % 2, (k + 1) % 4)

            wait_gather(k2, k)
            scale_fn(MSG[k2], AB[k])
            fire_scatter(k2, k)

    wait_scatter(1, 3)


def _zero_acc_rows(msg, acc_sh, r0):
    zero16 = jnp.zeros((16,), jnp.float32)

    @pl.loop(0, _CB)
    def _z(j):
        for hh in range(8):
            msg[j, pl.ds(hh * 16, 16)] = zero16

    for k in range(RPT // _CB):
        pltpu.sync_copy(msg.at[pl.ds(0, _CB)],
                        acc_sh.at[pl.ds(r0 + k * _CB, _CB)])


def _gatherg_body(srcdst, hin, g, sd0, sd1, sd2, sd3, msg0, msg1, isem, gsem,
                  ssem):
    c = lax.axis_index("c")
    s = lax.axis_index("s")
    base_t = (c * NS + s) * _EPW
    SD = [sd0, sd1, sd2, sd3]
    MSG = [msg0, msg1]
    nch = _EPW // _CB

    def issue_idx(ci, k):
        base = base_t + ci * _CB
        pltpu.async_copy(srcdst.at[0].at[pl.ds(base, _CB)], SD[k], isem)

    def wait_idx(k):
        pltpu.make_async_copy(srcdst.at[0].at[pl.ds(0, _CB)], SD[k],
                              isem).wait()

    def fire_gather(k2, k4):
        pltpu.async_copy(hin.at[SD[k4]], MSG[k2], gsem)

    def wait_gather(k2, k4):
        pltpu.make_async_copy(hin.at[SD[k4]], MSG[k2], gsem).wait()

    def fire_write(ci, k2):
        base = base_t + ci * _CB
        pltpu.async_copy(MSG[k2], g.at[pl.ds(base, _CB)], ssem)

    def wait_write(k2):
        pltpu.make_async_copy(MSG[k2], g.at[pl.ds(0, _CB)], ssem).wait()

    issue_idx(0, 0)
    issue_idx(1, 1)
    wait_idx(0)
    fire_gather(0, 0)

    @pl.loop(0, nch // 4)
    def _grp(g_):
        for k in range(4):
            ci = g_ * 4 + k
            k2 = k % 2

            @pl.when(ci >= 1)
            def _():
                wait_write((k - 1) % 2)

            @pl.when(ci + 2 < nch)
            def _():
                issue_idx(ci + 2, (k + 2) % 4)

            @pl.when(ci + 1 < nch)
            def _():
                wait_idx((k + 1) % 4)
                fire_gather((k + 1) % 2, (k + 1) % 4)

            wait_gather(k2, k)
            fire_write(ci, k2)

    wait_write(1)


def _gatherg_sc(srcdst, hin):
    return pl.kernel(
        _gatherg_body,
        out_type=jax.ShapeDtypeStruct((EP, D), jnp.float32),
        mesh=_mesh,
        compiler_params=pltpu.CompilerParams(needs_layout_passes=False),
        scratch_types=(
            [pltpu.VMEM((_CB,), jnp.int32)] * 4
            + [pltpu.VMEM((_CB, D), jnp.float32)] * 2
            + [pltpu.SemaphoreType.DMA] * 3
        ),
    )(srcdst, hin)


_EPW = EP // (NC * NS)


def _agg1_body(srcdst, alpha, h1, aggs, sd0, sd1, sd2, sd3, ab0, ab1, ab2,
               ab3, msg0, msg1, isem, gsem, ssem, acc_sh):
    c = lax.axis_index("c")
    s = lax.axis_index("s")
    r0 = s * RPT
    _zero_acc_rows(msg0, acc_sh, r0)
    plsc.subcore_barrier()

    def scale(msg, ab):
        @pl.loop(0, _CB, unroll=2)
        def _edge(j):
            jj = jnp.full((16,), j, jnp.int32)
            for hh in range(H):
                av = plsc.load_gather(ab, [jnp.full((16,), hh, jnp.int32), jj])
                sl = pl.ds(hh * 16, 16)
                msg[j, sl] = msg[j, sl] * av

    _agg_pipeline(srcdst, lambda ds: alpha.at[:, ds], h1, acc_sh,
                  [sd0, sd1, sd2, sd3], [ab0, ab1, ab2, ab3], [msg0, msg1],
                  isem, gsem, ssem, (c * NS + s) * _EPW, _EPW // _CB, scale)

    plsc.subcore_barrier()
    for k in range(RPT // _CB):
        sl = pl.ds(r0 + k * _CB, _CB)
        pltpu.sync_copy(acc_sh.at[sl], aggs.at[c].at[sl])


def _agg1_sc(srcdst, alpha, h1):
    return pl.kernel(
        _agg1_body,
        out_type=jax.ShapeDtypeStruct((2, NP, D), jnp.float32),
        mesh=_mesh,
        compiler_params=pltpu.CompilerParams(needs_layout_passes=False),
        scratch_types=(
            [pltpu.VMEM((2, _CB), jnp.int32)] * 4
            + [pltpu.VMEM((H, _CB), jnp.float32)] * 4
            + [pltpu.VMEM((_CB, D), jnp.float32)] * 2
            + [pltpu.SemaphoreType.DMA] * 3
            + [pltpu.VMEM_SHARED((NP, D), jnp.float32)]
        ),
    )(srcdst, alpha, h1)


def _agg2_body(srcdst, alpha, g, agg2, sd0, sd1, sd2, sd3, ab0, ab1, ab2,
               ab3, msg0, msg1, isem, gsem, ssem, acc_sh):
    c = lax.axis_index("c")
    s = lax.axis_index("s")
    r0 = s * RPT
    base_t = s * EPT

    def scale(msg, ab):
        @pl.loop(0, _CB // 16)
        def _grp(jg):
            j0 = jg * 16
            av16 = ab[pl.ds(j0, 16)]
            for e in range(16):
                av = av16[jnp.full((16,), e, jnp.int32)]
                for ch in range(8):
                    sl = pl.ds(ch * 16, 16)
                    msg[j0 + e, sl] = msg[j0 + e, sl] * av

    for p in range(HPC):
        _zero_acc_rows(msg0, acc_sh, r0)
        plsc.subcore_barrier()
        _agg_pipeline(srcdst, lambda ds: alpha.at[c * HPC + p].at[ds], g,
                      acc_sh, [sd0, sd1, sd2, sd3], [ab0, ab1, ab2, ab3],
                      [msg0, msg1], isem, gsem, ssem, base_t, EPT // _CB,
                      scale, linear_src=True)
        plsc.subcore_barrier()
        for k in range(RPT // _CB):
            sl = pl.ds(r0 + k * _CB, _CB)
            pltpu.sync_copy(acc_sh.at[sl], agg2.at[c].at[p].at[sl])


def _agg2_sc(srcdst, alpha, g):
    return pl.kernel(
        _agg2_body,
        out_type=jax.ShapeDtypeStruct((NC, HPC, NP, D), jnp.float32),
        mesh=_mesh,
        compiler_params=pltpu.CompilerParams(needs_layout_passes=False),
        scratch_types=(
            [pltpu.VMEM((2, _CB), jnp.int32)] * 4
            + [pltpu.VMEM((_CB,), jnp.float32)] * 4
            + [pltpu.VMEM((_CB, D), jnp.float32)] * 2
            + [pltpu.SemaphoreType.DMA] * 3
            + [pltpu.VMEM_SHARED((NP, D), jnp.float32)]
        ),
    )(srcdst, alpha, g)


def _fold_heads(a, dim):
    eye = jnp.eye(H, dtype=a.dtype)
    return (a[:, :, None] * eye[:, None, :]).reshape(H * dim, H)


def kernel(x, path_index, W1, a1_src, a1_dst, b1, W2, a2_src, a2_dst, b2):
    srcdst = jnp.pad(path_index.astype(jnp.int32), ((0, 0), (0, EP - E)),
                     constant_values=NP - 1)
    xp = jnp.pad(x, ((0, NP - N), (0, 0)))
    A1 = jnp.concatenate([_fold_heads(a1_src, 16), _fold_heads(a1_dst, 16)],
                         axis=1)
    A2 = jnp.concatenate([_fold_heads(a2_src, D), _fold_heads(a2_dst, D)],
                         axis=1)
    W2stack = W2.reshape(D, H, D).transpose(1, 0, 2)

    h1, st1 = _dense1(xp, W1, A1)
    alpha1 = _softmax_sc(st1.T, srcdst)
    aggs1 = _agg1_sc(srcdst, alpha1, h1)
    hin, st2 = _dense2(aggs1, b1, W2, A2)
    g = _gatherg_sc(srcdst, hin)
    alpha2 = _softmax_sc(st2.T, srcdst)
    agg2 = _agg2_sc(srcdst, alpha2, g)
    return _dense3(agg2, W2stack, b2)[:N]


# speedup vs baseline: 1.8980x; 1.0353x over previous
import jax
import jax.numpy as jnp
from jax import lax
from jax.experimental import pallas as pl
from jax.experimental.pallas import tpu as pltpu
from jax.experimental.pallas import tpu_sc as plsc

N = 10000
NP = 10240
E = 320000
EP = 327680
H = 8
D = 128
NC = 2
NS = 16
HPC = H // NC
EPT = EP // NS
RB = 1024
RPT = NP // NS

_mesh = plsc.VectorSubcoreMesh(core_axis_name="c", subcore_axis_name="s")


def _dense1_body(x_ref, w1_ref, a1_ref, h1_ref, st_ref):
    h1 = jnp.dot(x_ref[...], w1_ref[...], preferred_element_type=jnp.float32)
    h1_ref[...] = h1
    st_ref[...] = jnp.dot(h1, a1_ref[...], preferred_element_type=jnp.float32)


def _dense1(x, W1, A1):
    return pl.pallas_call(
        _dense1_body,
        grid=(NP // RB,),
        in_specs=[pl.BlockSpec((RB, D), lambda i: (i, 0)),
                  pl.BlockSpec((D, D), lambda i: (0, 0)),
                  pl.BlockSpec((D, 2 * H), lambda i: (0, 0))],
        out_specs=[pl.BlockSpec((RB, D), lambda i: (i, 0)),
                   pl.BlockSpec((RB, 2 * H), lambda i: (i, 0))],
        out_shape=[jax.ShapeDtypeStruct((NP, D), jnp.float32),
                   jax.ShapeDtypeStruct((NP, 2 * H), jnp.float32)],
    )(x, W1, A1)


def _dense2_body(agg_ref, b1_ref, w2_ref, a2_ref, hin_ref, st_ref):
    o = agg_ref[0] + agg_ref[1] + b1_ref[...]
    hin = jnp.where(o > 0, o, jnp.exp(o) - 1.0)
    hin_ref[...] = hin
    u2 = jnp.dot(w2_ref[...], a2_ref[...], preferred_element_type=jnp.float32)
    st_ref[...] = jnp.dot(hin, u2, preferred_element_type=jnp.float32)


def _dense2(aggs1, b1, W2, A2):
    return pl.pallas_call(
        _dense2_body,
        grid=(NP // RB,),
        in_specs=[pl.BlockSpec((2, RB, D), lambda i: (0, i, 0)),
                  pl.BlockSpec((1, D), lambda i: (0, 0)),
                  pl.BlockSpec((D, H * D), lambda i: (0, 0)),
                  pl.BlockSpec((H * D, 2 * H), lambda i: (0, 0))],
        out_specs=[pl.BlockSpec((RB, D), lambda i: (i, 0)),
                   pl.BlockSpec((RB, 2 * H), lambda i: (i, 0))],
        out_shape=[jax.ShapeDtypeStruct((NP, D), jnp.float32),
                   jax.ShapeDtypeStruct((NP, 2 * H), jnp.float32)],
    )(aggs1, b1.reshape(1, D), W2, A2)


def _dense3_body(agg_ref, w2s_ref, b2_ref, out_ref):
    acc = jnp.zeros((RB, D), jnp.float32)
    for c in range(NC):
        for p in range(HPC):
            acc = acc + jnp.dot(agg_ref[c, p], w2s_ref[c * HPC + p],
                                preferred_element_type=jnp.float32)
    out_ref[...] = acc * (1.0 / H) + b2_ref[...]


def _dense3(agg2, W2stack, b2):
    return pl.pallas_call(
        _dense3_body,
        grid=(NP // RB,),
        in_specs=[pl.BlockSpec((NC, HPC, RB, D), lambda i: (0, 0, i, 0)),
                  pl.BlockSpec((H, D, D), lambda i: (0, 0, 0)),
                  pl.BlockSpec((1, D), lambda i: (0, 0))],
        out_specs=pl.BlockSpec((RB, D), lambda i: (i, 0)),
        out_shape=jax.ShapeDtypeStruct((NP, D), jnp.float32),
    )(agg2, W2stack, b2.reshape(1, D))


_CH = 2048


def _softmax_body(stt, srcdst, alpha, s_vm, t_vm, d_vm,
                  sd0, sd1, sd2, sd3, ex0, ex1, ex2, ex3, isem, wsem,
                  den_sh):
    c = lax.axis_index("c")
    s = lax.axis_index("s")
    head = c * HPC + s // 4
    q = s % 4
    EQ = EP // 4
    base_q = q * EQ
    nch = EQ // _CH
    SD = [sd0, sd1, sd2, sd3]
    EX = [ex0, ex1, ex2, ex3]

    pltpu.sync_copy(stt.at[head], s_vm)
    pltpu.sync_copy(stt.at[H + head], t_vm)

    zero16 = jnp.zeros((16,), jnp.float32)

    @pl.loop(0, NP // 16)
    def _zero(i):
        d_vm[pl.ds(i * 16, 16)] = zero16

    def issue1(ci, k):
        base = base_q + ci * _CH
        pltpu.async_copy(srcdst.at[:, pl.ds(base, _CH)], SD[k], isem)

    def wait1(k):
        pltpu.make_async_copy(srcdst.at[:, pl.ds(0, _CH)], SD[k],
                              isem).wait()

    def fire_wr(ci, k):
        base = base_q + ci * _CH
        pltpu.async_copy(EX[k], alpha.at[head].at[pl.ds(base, _CH)], wsem)

    def wait_wr(k):
        pltpu.make_async_copy(EX[k], alpha.at[head].at[pl.ds(0, _CH)],
                              wsem).wait()

    issue1(0, 0)
    issue1(1, 1)

    @pl.loop(0, nch // 4)
    def _p1(g):
        for k in range(4):
            ci = g * 4 + k

            @pl.when(ci >= 1)
            def _():
                wait_wr((k - 1) % 4)

            @pl.when(ci + 2 < nch)
            def _():
                issue1(ci + 2, (k + 2) % 4)

            wait1(k)

            @pl.loop(0, _CH // 16, unroll=2)
            def _inner(j):
                sl = pl.ds(j * 16, 16)
                isrc = SD[k][0, sl]
                idst = SD[k][1, sl]
                e = (plsc.load_gather(s_vm, [isrc])
                     + plsc.load_gather(t_vm, [idst]))
                e = jnp.where(e >= 0.0, e, 0.2 * e)
                ex = jnp.exp(e)
                EX[k][sl] = ex
                plsc.addupdate_scatter(d_vm, [idst], ex)

            fire_wr(ci, k)

    wait_wr(3)

    pltpu.sync_copy(d_vm, den_sh.at[s])
    plsc.subcore_barrier()
    g4 = (s // 4) * 4

    pltpu.sync_copy(den_sh.at[g4], d_vm)
    for k in range(1, 4):
        pltpu.sync_copy(den_sh.at[g4 + k], t_vm)

        @pl.loop(0, NP // 16)
        def _acc(i):
            sl = pl.ds(i * 16, 16)
            d_vm[sl] = d_vm[sl] + t_vm[sl]

    @pl.loop(0, NP // 16)
    def _recip(i):
        sl = pl.ds(i * 16, 16)
        d_vm[sl] = 1.0 / (d_vm[sl] + 1e-16)

    def issue2(ci, k):
        base = base_q + ci * _CH
        pltpu.async_copy(srcdst.at[:, pl.ds(base, _CH)], SD[k], isem)
        pltpu.async_copy(alpha.at[head].at[pl.ds(base, _CH)], EX[k], isem)

    def wait2(k):
        pltpu.make_async_copy(srcdst.at[:, pl.ds(0, _CH)], SD[k],
                              isem).wait()
        pltpu.make_async_copy(alpha.at[head].at[pl.ds(0, _CH)], EX[k],
                              isem).wait()

    issue2(0, 0)
    issue2(1, 1)

    @pl.loop(0, nch // 4)
    def _p2(g):
        for k in range(4):
            ci = g * 4 + k

            @pl.when(ci >= 1)
            def _():
                wait_wr((k - 1) % 4)

            @pl.when(ci + 2 < nch)
            def _():
                issue2(ci + 2, (k + 2) % 4)

            wait2(k)

            @pl.loop(0, _CH // 16, unroll=2)
            def _inner(j):
                sl = pl.ds(j * 16, 16)
                rd = plsc.load_gather(d_vm, [SD[k][1, sl]])
                EX[k][sl] = EX[k][sl] * rd

            fire_wr(ci, k)

    wait_wr(3)


def _softmax_sc(stt, srcdst):
    return pl.kernel(
        _softmax_body,
        out_type=jax.ShapeDtypeStruct((H, EP), jnp.float32),
        mesh=_mesh,
        compiler_params=pltpu.CompilerParams(needs_layout_passes=False),
        scratch_types=(
            [pltpu.VMEM((NP,), jnp.float32)] * 3
            + [pltpu.VMEM((2, _CH), jnp.int32)] * 4
            + [pltpu.VMEM((_CH,), jnp.float32)] * 4
            + [pltpu.SemaphoreType.DMA] * 2
            + [pltpu.VMEM_SHARED((NS, NP), jnp.float32)]
        ),
    )(stt, srcdst)


_CB = 128


def _agg_pipeline(srcdst, alpha_slice, feat, acc_sh, SD, AB, MSG, isem, gsem,
                  ssem, base_t, nch, scale_fn, linear_src=False):
    def issue_idx(ci, k):
        base = base_t + ci * _CB
        pltpu.async_copy(srcdst.at[:, pl.ds(base, _CB)], SD[k], isem)
        pltpu.async_copy(alpha_slice(pl.ds(base, _CB)), AB[k], isem)

    def wait_idx(k):
        pltpu.make_async_copy(srcdst.at[:, pl.ds(0, _CB)], SD[k], isem).wait()
        pltpu.make_async_copy(alpha_slice(pl.ds(0, _CB)), AB[k], isem).wait()

    def fire_gather(ci, k2, k4):
        if linear_src:
            base = base_t + ci * _CB
            pltpu.async_copy(feat.at[pl.ds(base, _CB)], MSG[k2], gsem)
        else:
            pltpu.async_copy(feat.at[SD[k4].at[0]], MSG[k2], gsem)

    def wait_gather(k2, k4):
        if linear_src:
            pltpu.make_async_copy(feat.at[pl.ds(0, _CB)], MSG[k2],
                                  gsem).wait()
        else:
            pltpu.make_async_copy(feat.at[SD[k4].at[0]], MSG[k2],
                                  gsem).wait()

    def fire_scatter(k2, k4):
        pltpu.async_copy(MSG[k2], acc_sh.at[SD[k4].at[1]], ssem, add=True)

    def wait_scatter(k2, k4):
        pltpu.make_async_copy(MSG[k2], acc_sh.at[SD[k4].at[1]], ssem).wait()

    issue_idx(0, 0)
    issue_idx(1, 1)
    wait_idx(0)
    fire_gather(0, 0, 0)

    @pl.loop(0, nch // 4)
    def _grp(g):
        for k in range(4):
            ci = g * 4 + k
            k2 = k % 2

            @pl.when(ci >= 1)
            def _():
                wait_scatter((k - 1) % 2, (k - 1) % 4)

            @pl.when(ci + 2 < nch)
            def _():
                issue_idx(ci + 2, (k + 2) % 4)

            @pl.when(ci + 1 < nch)
            def _():
                wait_idx((k + 1) % 4)
                fire_gather(ci + 1, (k + 1) % 2, (k + 1) % 4)

            wait_gather(k2, k)
            scale_fn(MSG[k2], AB[k])
            fire_scatter(k2, k)

    wait_scatter(1, 3)


def _zero_acc_rows(msg, acc_sh, r0):
    zero16 = jnp.zeros((16,), jnp.float32)

    @pl.loop(0, _CB)
    def _z(j):
        for hh in range(8):
            msg[j, pl.ds(hh * 16, 16)] = zero16

    for k in range(RPT // _CB):
        pltpu.sync_copy(msg.at[pl.ds(0, _CB)],
                        acc_sh.at[pl.ds(r0 + k * _CB, _CB)])


def _gatherg_body(srcdst, hin, g, sd0, sd1, sd2, sd3, msg0, msg1, isem, gsem,
                  ssem):
    c = lax.axis_index("c")
    s = lax.axis_index("s")
    base_t = (c * NS + s) * _EPW
    SD = [sd0, sd1, sd2, sd3]
    MSG = [msg0, msg1]
    nch = _EPW // _CB

    def issue_idx(ci, k):
        base = base_t + ci * _CB
        pltpu.async_copy(srcdst.at[0].at[pl.ds(base, _CB)], SD[k], isem)

    def wait_idx(k):
        pltpu.make_async_copy(srcdst.at[0].at[pl.ds(0, _CB)], SD[k],
                              isem).wait()

    def fire_gather(k2, k4):
        pltpu.async_copy(hin.at[SD[k4]], MSG[k2], gsem)

    def wait_gather(k2, k4):
        pltpu.make_async_copy(hin.at[SD[k4]], MSG[k2], gsem).wait()

    def fire_write(ci, k2):
        base = base_t + ci * _CB
        pltpu.async_copy(MSG[k2], g.at[pl.ds(base, _CB)], ssem)

    def wait_write(k2):
        pltpu.make_async_copy(MSG[k2], g.at[pl.ds(0, _CB)], ssem).wait()

    issue_idx(0, 0)
    issue_idx(1, 1)
    wait_idx(0)
    fire_gather(0, 0)

    @pl.loop(0, nch // 4)
    def _grp(g_):
        for k in range(4):
            ci = g_ * 4 + k
            k2 = k % 2

            @pl.when(ci >= 1)
            def _():
                wait_write((k - 1) % 2)

            @pl.when(ci + 2 < nch)
            def _():
                issue_idx(ci + 2, (k + 2) % 4)

            @pl.when(ci + 1 < nch)
            def _():
                wait_idx((k + 1) % 4)
                fire_gather((k + 1) % 2, (k + 1) % 4)

            wait_gather(k2, k)
            fire_write(ci, k2)

    wait_write(1)


def _gatherg_sc(srcdst, hin):
    return pl.kernel(
        _gatherg_body,
        out_type=jax.ShapeDtypeStruct((EP, D), jnp.float32),
        mesh=_mesh,
        compiler_params=pltpu.CompilerParams(needs_layout_passes=False),
        scratch_types=(
            [pltpu.VMEM((_CB,), jnp.int32)] * 4
            + [pltpu.VMEM((_CB, D), jnp.float32)] * 2
            + [pltpu.SemaphoreType.DMA] * 3
        ),
    )(srcdst, hin)


_EPW = EP // (NC * NS)


def _agg1_body(srcdst, alpha, h1, aggs, sd0, sd1, sd2, sd3, ab0, ab1, ab2,
               ab3, msg0, msg1, isem, gsem, ssem, acc_sh):
    c = lax.axis_index("c")
    s = lax.axis_index("s")
    r0 = s * RPT
    _zero_acc_rows(msg0, acc_sh, r0)
    plsc.subcore_barrier()

    def scale(msg, ab):
        @pl.loop(0, _CB // 16)
        def _grp(jg):
            j0 = jg * 16
            avh = [ab[hh, pl.ds(j0, 16)] for hh in range(H)]
            for e in range(16):
                idx = jnp.full((16,), e, jnp.int32)
                for hh in range(H):
                    sl = pl.ds(hh * 16, 16)
                    msg[j0 + e, sl] = msg[j0 + e, sl] * avh[hh][idx]

    _agg_pipeline(srcdst, lambda ds: alpha.at[:, ds], h1, acc_sh,
                  [sd0, sd1, sd2, sd3], [ab0, ab1, ab2, ab3], [msg0, msg1],
                  isem, gsem, ssem, (c * NS + s) * _EPW, _EPW // _CB, scale)

    plsc.subcore_barrier()
    for k in range(RPT // _CB):
        sl = pl.ds(r0 + k * _CB, _CB)
        pltpu.sync_copy(acc_sh.at[sl], aggs.at[c].at[sl])


def _agg1_sc(srcdst, alpha, h1):
    return pl.kernel(
        _agg1_body,
        out_type=jax.ShapeDtypeStruct((2, NP, D), jnp.float32),
        mesh=_mesh,
        compiler_params=pltpu.CompilerParams(needs_layout_passes=False),
        scratch_types=(
            [pltpu.VMEM((2, _CB), jnp.int32)] * 4
            + [pltpu.VMEM((H, _CB), jnp.float32)] * 4
            + [pltpu.VMEM((_CB, D), jnp.float32)] * 2
            + [pltpu.SemaphoreType.DMA] * 3
            + [pltpu.VMEM_SHARED((NP, D), jnp.float32)]
        ),
    )(srcdst, alpha, h1)


def _agg2_body(srcdst, alpha, g, agg2, sd0, sd1, sd2, sd3, ab0, ab1, ab2,
               ab3, msg0, msg1, isem, gsem, ssem, acc_sh):
    c = lax.axis_index("c")
    s = lax.axis_index("s")
    r0 = s * RPT
    base_t = s * EPT

    def scale(msg, ab):
        @pl.loop(0, _CB // 16)
        def _grp(jg):
            j0 = jg * 16
            av16 = ab[pl.ds(j0, 16)]
            for e in range(16):
                av = av16[jnp.full((16,), e, jnp.int32)]
                for ch in range(8):
                    sl = pl.ds(ch * 16, 16)
                    msg[j0 + e, sl] = msg[j0 + e, sl] * av

    for p in range(HPC):
        _zero_acc_rows(msg0, acc_sh, r0)
        plsc.subcore_barrier()
        _agg_pipeline(srcdst, lambda ds: alpha.at[c * HPC + p].at[ds], g,
                      acc_sh, [sd0, sd1, sd2, sd3], [ab0, ab1, ab2, ab3],
                      [msg0, msg1], isem, gsem, ssem, base_t, EPT // _CB,
                      scale, linear_src=True)
        plsc.subcore_barrier()
        for k in range(RPT // _CB):
            sl = pl.ds(r0 + k * _CB, _CB)
            pltpu.sync_copy(acc_sh.at[sl], agg2.at[c].at[p].at[sl])


def _agg2_sc(srcdst, alpha, g):
    return pl.kernel(
        _agg2_body,
        out_type=jax.ShapeDtypeStruct((NC, HPC, NP, D), jnp.float32),
        mesh=_mesh,
        compiler_params=pltpu.CompilerParams(needs_layout_passes=False),
        scratch_types=(
            [pltpu.VMEM((2, _CB), jnp.int32)] * 4
            + [pltpu.VMEM((_CB,), jnp.float32)] * 4
            + [pltpu.VMEM((_CB, D), jnp.float32)] * 2
            + [pltpu.SemaphoreType.DMA] * 3
            + [pltpu.VMEM_SHARED((NP, D), jnp.float32)]
        ),
    )(srcdst, alpha, g)


def _fold_heads(a, dim):
    eye = jnp.eye(H, dtype=a.dtype)
    return (a[:, :, None] * eye[:, None, :]).reshape(H * dim, H)


def kernel(x, path_index, W1, a1_src, a1_dst, b1, W2, a2_src, a2_dst, b2):
    srcdst = jnp.pad(path_index.astype(jnp.int32), ((0, 0), (0, EP - E)),
                     constant_values=NP - 1)
    xp = jnp.pad(x, ((0, NP - N), (0, 0)))
    A1 = jnp.concatenate([_fold_heads(a1_src, 16), _fold_heads(a1_dst, 16)],
                         axis=1)
    A2 = jnp.concatenate([_fold_heads(a2_src, D), _fold_heads(a2_dst, D)],
                         axis=1)
    W2stack = W2.reshape(D, H, D).transpose(1, 0, 2)

    h1, st1 = _dense1(xp, W1, A1)
    alpha1 = _softmax_sc(st1.T, srcdst)
    aggs1 = _agg1_sc(srcdst, alpha1, h1)
    hin, st2 = _dense2(aggs1, b1, W2, A2)
    g = _gatherg_sc(srcdst, hin)
    alpha2 = _softmax_sc(st2.T, srcdst)
    agg2 = _agg2_sc(srcdst, alpha2, g)
    return _dense3(agg2, W2stack, b2)[:N]
